# Initial kernel scaffold; baseline (speedup 1.0000x reference)
import functools

import jax
import jax.numpy as jnp
from jax import lax
from jax.experimental import pallas as pl
from jax.experimental.pallas import tpu as pltpu
from jax.experimental.pallas import tpu_sc as plsc

_N = 10000
_E = 320000
_D = 128
_NPAD = 10240
_RPS = _NPAD // 16
_B = 128
_CPW = 80
_NBUF = 4
_GROUPS = _CPW // _NBUF
_NW = 32
_CHUNKS = _NW * _CPW
_EPAD = _CHUNKS * _B
_RB = 1024

_mesh = plsc.VectorSubcoreMesh(core_axis_name="c", subcore_axis_name="s")


@functools.partial(
    pl.kernel,
    out_type=jax.ShapeDtypeStruct((2, _NPAD, 1), jnp.float32),
    mesh=_mesh,
    scratch_types=[
        pltpu.VMEM((_CPW, _B), jnp.int32),
        pltpu.VMEM((_B, 1), jnp.float32),
        pltpu.VMEM_SHARED((_NPAD, 1), jnp.float32),
        pltpu.SemaphoreType.DMA((_NBUF,)),
    ],
)
def _deg_kernel(dst_hbm, ones_hbm, zcol_hbm, out_hbm, didx, ones_v, acc, sem):
    cid = lax.axis_index("c")
    sid = lax.axis_index("s")
    base = (cid * 16 + sid) * _CPW
    r0 = sid * _RPS
    pltpu.sync_copy(dst_hbm.at[pl.ds(base, _CPW)], didx)
    pltpu.sync_copy(ones_hbm, ones_v)
    pltpu.sync_copy(zcol_hbm, acc.at[pl.ds(r0, _RPS)])
    plsc.subcore_barrier()

    def body(g, carry):
        for b in range(_NBUF):
            jj = g * _NBUF + b
            pltpu.async_copy(ones_v, acc.at[didx.at[jj]], sem.at[b], add=True)
        for b in range(_NBUF):
            jj = g * _NBUF + b
            pltpu.make_async_copy(ones_v, acc.at[didx.at[jj]], sem.at[b]).wait()
        return carry

    lax.fori_loop(0, _GROUPS, body, 0)
    plsc.subcore_barrier()
    pltpu.sync_copy(acc.at[pl.ds(r0, _RPS)],
                    out_hbm.at[cid].at[pl.ds(r0, _RPS)])


@functools.partial(
    pl.kernel,
    out_type=jax.ShapeDtypeStruct((2, _NPAD, _D), jnp.float32),
    mesh=_mesh,
    scratch_types=[
        pltpu.VMEM((_CPW, _B), jnp.int32),
        pltpu.VMEM((_CPW, _B), jnp.int32),
        pltpu.VMEM((_NBUF, _B, _D), jnp.float32),
        pltpu.VMEM_SHARED((_NPAD, _D), jnp.float32),
        pltpu.SemaphoreType.DMA((_NBUF,)),
        pltpu.SemaphoreType.DMA((_NBUF,)),
    ],
)
def _agg_kernel(hs_hbm, src_hbm, dst_hbm, zrow_hbm, out_hbm,
                sidx, didx, rbuf, acc, gsem, ssem):
    cid = lax.axis_index("c")
    sid = lax.axis_index("s")
    base = (cid * 16 + sid) * _CPW
    r0 = sid * _RPS
    pltpu.sync_copy(src_hbm.at[pl.ds(base, _CPW)], sidx)
    pltpu.sync_copy(dst_hbm.at[pl.ds(base, _CPW)], didx)
    pltpu.sync_copy(zrow_hbm, acc.at[pl.ds(r0, _RPS)])
    plsc.subcore_barrier()

    for b in range(_NBUF):
        pltpu.async_copy(hs_hbm.at[sidx.at[b]], rbuf.at[b], gsem.at[b])

    def body(g, carry):
        for b in range(_NBUF):
            jj = g * _NBUF + b
            pltpu.make_async_copy(hs_hbm.at[sidx.at[jj]], rbuf.at[b],
                                  gsem.at[b]).wait()
            pltpu.async_copy(rbuf.at[b], acc.at[didx.at[jj]], ssem.at[b],
                             add=True)
        for b in range(_NBUF):
            jj = g * _NBUF + b
            pltpu.make_async_copy(rbuf.at[b], acc.at[didx.at[jj]],
                                  ssem.at[b]).wait()

            @pl.when(g < _GROUPS - 1)
            def _():
                pltpu.async_copy(hs_hbm.at[sidx.at[jj + _NBUF]], rbuf.at[b],
                                 gsem.at[b])
        return carry

    lax.fori_loop(0, _GROUPS, body, 0)
    plsc.subcore_barrier()
    pltpu.sync_copy(acc.at[pl.ds(r0, _RPS)],
                    out_hbm.at[cid].at[pl.ds(r0, _RPS)])


def _prep_body(x_ref, w_ref, degp_ref, out_ref):
    dinv = lax.rsqrt(degp_ref[0] + degp_ref[1] + 1.0)
    h = jnp.dot(x_ref[...], w_ref[...], preferred_element_type=jnp.float32)
    out_ref[...] = h * dinv


def _prep(x_pad, w, degp):
    return pl.pallas_call(
        _prep_body,
        grid=(_NPAD // _RB,),
        in_specs=[
            pl.BlockSpec((_RB, _D), lambda i: (i, 0)),
            pl.BlockSpec((_D, _D), lambda i: (0, 0)),
            pl.BlockSpec((2, _RB, 1), lambda i: (0, i, 0)),
        ],
        out_specs=pl.BlockSpec((_RB, _D), lambda i: (i, 0)),
        out_shape=jax.ShapeDtypeStruct((_NPAD, _D), jnp.float32),
    )(x_pad, w, degp)


def _layer_body(aggp_ref, hs_ref, degp_ref, b_ref, w_ref, out_ref):
    dinv = lax.rsqrt(degp_ref[0] + degp_ref[1] + 1.0)
    z = (aggp_ref[0] + aggp_ref[1] + hs_ref[...]) * dinv + b_ref[...]
    z = jnp.maximum(z, 0.0)
    out_ref[...] = jnp.dot(z, w_ref[...],
                           preferred_element_type=jnp.float32) * dinv


def _layer(aggp, hs, degp, b, w):
    return pl.pallas_call(
        _layer_body,
        grid=(_NPAD // _RB,),
        in_specs=[
            pl.BlockSpec((2, _RB, _D), lambda i: (0, i, 0)),
            pl.BlockSpec((_RB, _D), lambda i: (i, 0)),
            pl.BlockSpec((2, _RB, 1), lambda i: (0, i, 0)),
            pl.BlockSpec((1, _D), lambda i: (0, 0)),
            pl.BlockSpec((_D, _D), lambda i: (0, 0)),
        ],
        out_specs=pl.BlockSpec((_RB, _D), lambda i: (i, 0)),
        out_shape=jax.ShapeDtypeStruct((_NPAD, _D), jnp.float32),
    )(aggp, hs, degp, b, w)


def _final_body(aggp_ref, hs_ref, degp_ref, b_ref, wfc_ref, bfc_ref,
                out_ref, acc_ref):
    i = pl.program_id(0)
    dinv = lax.rsqrt(degp_ref[0] + degp_ref[1] + 1.0)
    z = (aggp_ref[0] + aggp_ref[1] + hs_ref[...]) * dinv + b_ref[...]
    z = jnp.maximum(z, 0.0)
    rid = i * _RB + lax.broadcasted_iota(jnp.int32, (_RB, 1), 0)
    z = jnp.where(rid < _N, z, 0.0)

    @pl.when(i == 0)
    def _():
        acc_ref[...] = jnp.zeros_like(acc_ref)

    acc_ref[...] += jnp.sum(z, axis=0, keepdims=True)

    @pl.when(i == pl.num_programs(0) - 1)
    def _():
        g = acc_ref[...] * (1.0 / _N)
        out_ref[...] = jnp.dot(g, wfc_ref[...],
                               preferred_element_type=jnp.float32) + bfc_ref[...]


def _final(aggp, hs, degp, b, wfc, bfc):
    return pl.pallas_call(
        _final_body,
        grid=(_NPAD // _RB,),
        in_specs=[
            pl.BlockSpec((2, _RB, _D), lambda i: (0, i, 0)),
            pl.BlockSpec((_RB, _D), lambda i: (i, 0)),
            pl.BlockSpec((2, _RB, 1), lambda i: (0, i, 0)),
            pl.BlockSpec((1, _D), lambda i: (0, 0)),
            pl.BlockSpec((_D, 1), lambda i: (0, 0)),
            pl.BlockSpec((1, 1), lambda i: (0, 0)),
        ],
        out_specs=pl.BlockSpec((1, 1), lambda i: (0, 0)),
        out_shape=jax.ShapeDtypeStruct((1, 1), jnp.float32),
        scratch_shapes=[pltpu.VMEM((1, _D), jnp.float32)],
    )(aggp, hs, degp, b, wfc, bfc)


def kernel(x, edge_index, W1, b1, W2, b2, Wfc, bfc):
    src = edge_index[0]
    dst = edge_index[1]
    pad = (jnp.arange(_EPAD - _E, dtype=jnp.int32) % 128) + _N
    src3 = jnp.concatenate([src, pad]).reshape(_CHUNKS, _B)
    dst3 = jnp.concatenate([dst, pad]).reshape(_CHUNKS, _B)
    x_pad = jnp.zeros((_NPAD, _D), jnp.float32).at[:_N].set(x)
    zrow = jnp.zeros((_RPS, _D), jnp.float32)
    zcol = jnp.zeros((_RPS, 1), jnp.float32)
    ones = jnp.ones((_B, 1), jnp.float32)

    degp = _deg_kernel(dst3, ones, zcol)
    hs1 = _prep(x_pad, W1, degp)
    agg1 = _agg_kernel(hs1, src3, dst3, zrow)
    hs2 = _layer(agg1, hs1, degp, b1.reshape(1, _D), W2)
    agg2 = _agg_kernel(hs2, src3, dst3, zrow)
    out = _final(agg2, hs2, degp, b2.reshape(1, _D), Wfc, bfc.reshape(1, 1))
    return out.reshape((1,))


# speedup vs baseline: 15.4941x; 15.4941x over previous
import functools

import jax
import jax.numpy as jnp
from jax import lax
from jax.experimental import pallas as pl
from jax.experimental.pallas import tpu as pltpu
from jax.experimental.pallas import tpu_sc as plsc

_N = 10000
_E = 320000
_D = 128
_NPAD = 10240
_RPS = _NPAD // 16
_B = 128
_CPW = 80
_NBUF = 4
_GROUPS = _CPW // _NBUF
_NW = 32
_CHUNKS = _NW * _CPW
_EPAD = _CHUNKS * _B
_RB = 1024

_mesh = plsc.VectorSubcoreMesh(core_axis_name="c", subcore_axis_name="s")
_mesh1 = plsc.VectorSubcoreMesh(core_axis_name="c", subcore_axis_name="s",
                                num_cores=1)


@functools.partial(
    pl.kernel,
    out_type=jax.ShapeDtypeStruct((2, _NPAD, 1), jnp.float32),
    mesh=_mesh,
    scratch_types=[
        pltpu.VMEM((_CPW, _B), jnp.int32),
        pltpu.VMEM((_B, 1), jnp.float32),
        pltpu.VMEM_SHARED((_NPAD, 1), jnp.float32),
        pltpu.SemaphoreType.DMA((_NBUF,)),
    ],
)
def _deg_kernel(dst_hbm, ones_hbm, zcol_hbm, out_hbm, didx, ones_v, acc, sem):
    cid = lax.axis_index("c")
    sid = lax.axis_index("s")
    base = (cid * 16 + sid) * _CPW
    r0 = sid * _RPS
    pltpu.sync_copy(dst_hbm.at[pl.ds(base, _CPW)], didx)
    pltpu.sync_copy(ones_hbm, ones_v)
    pltpu.sync_copy(zcol_hbm, acc.at[pl.ds(r0, _RPS)])
    plsc.subcore_barrier()

    def body(g, carry):
        for b in range(_NBUF):
            jj = g * _NBUF + b
            pltpu.async_copy(ones_v, acc.at[didx.at[jj]], sem.at[b], add=True)
        for b in range(_NBUF):
            jj = g * _NBUF + b
            pltpu.make_async_copy(ones_v, acc.at[didx.at[jj]], sem.at[b]).wait()
        return carry

    lax.fori_loop(0, _GROUPS, body, 0)
    plsc.subcore_barrier()
    pltpu.sync_copy(acc.at[pl.ds(r0, _RPS)],
                    out_hbm.at[cid].at[pl.ds(r0, _RPS)])


_CPW_A = _CHUNKS // 16
_NQ = 4
_QCH = _CPW_A // _NQ
_NBUF_A = 2
_GROUPS_Q = _QCH // _NBUF_A


@functools.partial(
    pl.kernel,
    out_type=jax.ShapeDtypeStruct((_NPAD, _D), jnp.float32),
    mesh=_mesh1,
    scratch_types=[
        pltpu.VMEM((_QCH, _B), jnp.int32),
        pltpu.VMEM((_QCH, _B), jnp.int32),
        pltpu.VMEM((_NBUF_A, _B, _D), jnp.float32),
        pltpu.VMEM_SHARED((_NPAD, _D), jnp.float32),
        pltpu.SemaphoreType.DMA((_NBUF_A,)),
        pltpu.SemaphoreType.DMA((_NBUF_A,)),
    ],
)
def _agg_kernel(hs_hbm, src_hbm, dst_hbm, zrow_hbm, out_hbm,
                sidx, didx, rbuf, acc, gsem, ssem):
    sid = lax.axis_index("s")
    base = sid * _CPW_A
    r0 = sid * _RPS
    pltpu.sync_copy(zrow_hbm, acc.at[pl.ds(r0, _RPS)])
    plsc.subcore_barrier()

    def quarter(q, carry):
        qb = base + q * _QCH
        pltpu.sync_copy(src_hbm.at[pl.ds(qb, _QCH)], sidx)
        pltpu.sync_copy(dst_hbm.at[pl.ds(qb, _QCH)], didx)

        for b in range(_NBUF_A):
            pltpu.async_copy(hs_hbm.at[sidx.at[b]], rbuf.at[b], gsem.at[b])

        def body(g, carry2):
            for b in range(_NBUF_A):
                jj = g * _NBUF_A + b
                pltpu.make_async_copy(hs_hbm.at[sidx.at[jj]], rbuf.at[b],
                                      gsem.at[b]).wait()
                pltpu.async_copy(rbuf.at[b], acc.at[didx.at[jj]], ssem.at[b],
                                 add=True)
            for b in range(_NBUF_A):
                jj = g * _NBUF_A + b
                pltpu.make_async_copy(rbuf.at[b], acc.at[didx.at[jj]],
                                      ssem.at[b]).wait()

                @pl.when(g < _GROUPS_Q - 1)
                def _():
                    pltpu.async_copy(hs_hbm.at[sidx.at[jj + _NBUF_A]],
                                     rbuf.at[b], gsem.at[b])
            return carry2

        lax.fori_loop(0, _GROUPS_Q, body, 0)
        return carry

    lax.fori_loop(0, _NQ, quarter, 0)
    plsc.subcore_barrier()
    pltpu.sync_copy(acc.at[pl.ds(r0, _RPS)], out_hbm.at[pl.ds(r0, _RPS)])


def _prep_body(x_ref, w_ref, degp_ref, out_ref):
    dinv = lax.rsqrt(degp_ref[0] + degp_ref[1] + 1.0)
    h = jnp.dot(x_ref[...], w_ref[...],
                preferred_element_type=jnp.float32)
    out_ref[...] = h * dinv


def _prep(x_pad, w, degp):
    return pl.pallas_call(
        _prep_body,
        grid=(_NPAD // _RB,),
        in_specs=[
            pl.BlockSpec((_RB, _D), lambda i: (i, 0)),
            pl.BlockSpec((_D, _D), lambda i: (0, 0)),
            pl.BlockSpec((2, _RB, 1), lambda i: (0, i, 0)),
        ],
        out_specs=pl.BlockSpec((_RB, _D), lambda i: (i, 0)),
        out_shape=jax.ShapeDtypeStruct((_NPAD, _D), jnp.float32),
    )(x_pad, w, degp)


def _layer_body(agg_ref, hs_ref, degp_ref, b_ref, w_ref, out_ref):
    dinv = lax.rsqrt(degp_ref[0] + degp_ref[1] + 1.0)
    z = (agg_ref[...] + hs_ref[...]) * dinv + b_ref[...]
    z = jnp.maximum(z, 0.0)
    out_ref[...] = jnp.dot(z, w_ref[...],
                           preferred_element_type=jnp.float32) * dinv


def _layer(agg, hs, degp, b, w):
    return pl.pallas_call(
        _layer_body,
        grid=(_NPAD // _RB,),
        in_specs=[
            pl.BlockSpec((_RB, _D), lambda i: (i, 0)),
            pl.BlockSpec((_RB, _D), lambda i: (i, 0)),
            pl.BlockSpec((2, _RB, 1), lambda i: (0, i, 0)),
            pl.BlockSpec((1, _D), lambda i: (0, 0)),
            pl.BlockSpec((_D, _D), lambda i: (0, 0)),
        ],
        out_specs=pl.BlockSpec((_RB, _D), lambda i: (i, 0)),
        out_shape=jax.ShapeDtypeStruct((_NPAD, _D), jnp.float32),
    )(agg, hs, degp, b, w)


def _final_body(agg_ref, hs_ref, degp_ref, b_ref, wfc_ref, bfc_ref,
                out_ref, acc_ref):
    i = pl.program_id(0)
    dinv = lax.rsqrt(degp_ref[0] + degp_ref[1] + 1.0)
    z = (agg_ref[...] + hs_ref[...]) * dinv + b_ref[...]
    z = jnp.maximum(z, 0.0)
    rid = i * _RB + lax.broadcasted_iota(jnp.int32, (_RB, 1), 0)
    z = jnp.where(rid < _N, z, 0.0)

    @pl.when(i == 0)
    def _():
        acc_ref[...] = jnp.zeros_like(acc_ref)

    acc_ref[...] += jnp.sum(z, axis=0, keepdims=True)

    @pl.when(i == pl.num_programs(0) - 1)
    def _():
        g = acc_ref[...] * (1.0 / _N)
        out_ref[...] = jnp.dot(g, wfc_ref[...],
                               preferred_element_type=jnp.float32) + bfc_ref[...]


def _final(agg, hs, degp, b, wfc, bfc):
    return pl.pallas_call(
        _final_body,
        grid=(_NPAD // _RB,),
        in_specs=[
            pl.BlockSpec((_RB, _D), lambda i: (i, 0)),
            pl.BlockSpec((_RB, _D), lambda i: (i, 0)),
            pl.BlockSpec((2, _RB, 1), lambda i: (0, i, 0)),
            pl.BlockSpec((1, _D), lambda i: (0, 0)),
            pl.BlockSpec((_D, 1), lambda i: (0, 0)),
            pl.BlockSpec((1, 1), lambda i: (0, 0)),
        ],
        out_specs=pl.BlockSpec((1, 1), lambda i: (0, 0)),
        out_shape=jax.ShapeDtypeStruct((1, 1), jnp.float32),
        scratch_shapes=[pltpu.VMEM((1, _D), jnp.float32)],
    )(agg, hs, degp, b, wfc, bfc)


def kernel(x, edge_index, W1, b1, W2, b2, Wfc, bfc):
    src = edge_index[0]
    dst = edge_index[1]
    pad = (jnp.arange(_EPAD - _E, dtype=jnp.int32) % 128) + _N
    src3 = jnp.concatenate([src, pad]).reshape(_CHUNKS, _B)
    dst3 = jnp.concatenate([dst, pad]).reshape(_CHUNKS, _B)
    x_pad = jnp.zeros((_NPAD, _D), jnp.float32).at[:_N].set(x)
    zrow = jnp.zeros((_RPS, _D), jnp.float32)
    zcol = jnp.zeros((_RPS, 1), jnp.float32)
    ones = jnp.ones((_B, 1), jnp.float32)

    degp = _deg_kernel(dst3, ones, zcol)
    hs1 = _prep(x_pad, W1, degp)
    agg1 = _agg_kernel(hs1, src3, dst3, zrow)
    hs2 = _layer(agg1, hs1, degp, b1.reshape(1, _D), W2)
    agg2 = _agg_kernel(hs2, src3, dst3, zrow)
    out = _final(agg2, hs2, degp, b2.reshape(1, _D), Wfc, bfc.reshape(1, 1))
    return out.reshape((1,))


# speedup vs baseline: 16.0029x; 1.0328x over previous
import functools

import jax
import jax.numpy as jnp
from jax import lax
from jax.experimental import pallas as pl
from jax.experimental.pallas import tpu as pltpu
from jax.experimental.pallas import tpu_sc as plsc

_N = 10000
_E = 320000
_D = 128
_NPAD = 10240
_RPS = _NPAD // 16
_B = 128
_CPW = 80
_NBUF = 4
_GROUPS = _CPW // _NBUF
_NW = 32
_CHUNKS = _NW * _CPW
_EPAD = _CHUNKS * _B
_RB = 1024

_mesh = plsc.VectorSubcoreMesh(core_axis_name="c", subcore_axis_name="s")
_mesh1 = plsc.VectorSubcoreMesh(core_axis_name="c", subcore_axis_name="s",
                                num_cores=1)


_CPW_A = _CHUNKS // 16
_NQ = 4
_QCH = _CPW_A // _NQ
_NBUF_A = 4
_GROUPS_Q = _QCH // _NBUF_A
_NC_CORE = 5120
_GZ = 1024
_NLOC = _NC_CORE + _GZ
_ZPT = _NLOC // 16
_WPT = _NC_CORE // 16


@functools.partial(
    pl.kernel,
    out_type=jax.ShapeDtypeStruct((_NPAD, _D), jnp.float32),
    mesh=_mesh,
    scratch_types=[
        pltpu.VMEM((_CHUNKS // 16 // 4, _B), jnp.int32),
        pltpu.VMEM((_B, _D), jnp.float32),
        pltpu.VMEM_SHARED((_NLOC, _D), jnp.float32),
        pltpu.SemaphoreType.DMA((_NBUF,)),
    ],
)
def _deg_kernel(dloc_hbm, ones_hbm, zrow_hbm, out_hbm, didx, ones_v, acc, sem):
    cid = lax.axis_index("c")
    sid = lax.axis_index("s")
    base = sid * (_CHUNKS // 16)
    qch = _CHUNKS // 16 // 4
    dlo = cid * _NC_CORE
    pltpu.sync_copy(ones_hbm, ones_v)
    pltpu.sync_copy(zrow_hbm, acc.at[pl.ds(sid * _ZPT, _ZPT)])
    plsc.subcore_barrier()

    def quarter(q, carry):
        pltpu.sync_copy(dloc_hbm.at[cid].at[pl.ds(base + q * qch, qch)], didx)

        def body(g, carry2):
            for b in range(_NBUF):
                jj = g * _NBUF + b
                pltpu.async_copy(ones_v, acc.at[didx.at[jj]], sem.at[b],
                                 add=True)
            for b in range(_NBUF):
                jj = g * _NBUF + b
                pltpu.make_async_copy(ones_v, acc.at[didx.at[jj]],
                                      sem.at[b]).wait()
            return carry2

        lax.fori_loop(0, qch // _NBUF, body, 0)
        return carry

    lax.fori_loop(0, 4, quarter, 0)
    plsc.subcore_barrier()
    pltpu.sync_copy(acc.at[pl.ds(sid * _WPT, _WPT)],
                    out_hbm.at[pl.ds(dlo + sid * _WPT, _WPT)])


@functools.partial(
    pl.kernel,
    out_type=jax.ShapeDtypeStruct((_NPAD, _D), jnp.float32),
    mesh=_mesh,
    scratch_types=[
        pltpu.VMEM((_QCH, _B), jnp.int32),
        pltpu.VMEM((_QCH, _B), jnp.int32),
        pltpu.VMEM((_NBUF_A, _B, _D), jnp.float32),
        pltpu.VMEM_SHARED((_NLOC, _D), jnp.float32),
        pltpu.SemaphoreType.DMA((_NBUF_A,)),
        pltpu.SemaphoreType.DMA((_NBUF_A,)),
    ],
)
def _agg_kernel(hs_hbm, src_hbm, dloc_hbm, zrow_hbm, out_hbm,
                sidx, didx, rbuf, acc, gsem, ssem):
    cid = lax.axis_index("c")
    sid = lax.axis_index("s")
    base = sid * _CPW_A
    dlo = cid * _NC_CORE
    pltpu.sync_copy(zrow_hbm, acc.at[pl.ds(sid * _ZPT, _ZPT)])
    plsc.subcore_barrier()

    def quarter(q, carry):
        qb = base + q * _QCH
        pltpu.sync_copy(src_hbm.at[pl.ds(qb, _QCH)], sidx)
        pltpu.sync_copy(dloc_hbm.at[cid].at[pl.ds(qb, _QCH)], didx)

        for b in range(_NBUF_A):
            pltpu.async_copy(hs_hbm.at[sidx.at[b]], rbuf.at[b], gsem.at[b])

        def body(g, carry2):
            for b in range(_NBUF_A):
                jj = g * _NBUF_A + b
                pltpu.make_async_copy(hs_hbm.at[sidx.at[jj]], rbuf.at[b],
                                      gsem.at[b]).wait()
                pltpu.async_copy(rbuf.at[b], acc.at[didx.at[jj]], ssem.at[b],
                                 add=True)
            for b in range(_NBUF_A):
                jj = g * _NBUF_A + b
                pltpu.make_async_copy(rbuf.at[b], acc.at[didx.at[jj]],
                                      ssem.at[b]).wait()

                @pl.when(g < _GROUPS_Q - 1)
                def _(jj=jj, b=b):
                    pltpu.async_copy(hs_hbm.at[sidx.at[jj + _NBUF_A]],
                                     rbuf.at[b], gsem.at[b])
            return carry2

        lax.fori_loop(0, _GROUPS_Q, body, 0)
        return carry

    lax.fori_loop(0, _NQ, quarter, 0)
    plsc.subcore_barrier()
    pltpu.sync_copy(acc.at[pl.ds(sid * _WPT, _WPT)],
                    out_hbm.at[pl.ds(dlo + sid * _WPT, _WPT)])


def _prep_body(x_ref, w_ref, degc_ref, out_ref):
    dinv = lax.rsqrt(degc_ref[...] + 1.0)
    h = jnp.dot(x_ref[...], w_ref[...],
                preferred_element_type=jnp.float32)
    out_ref[...] = h * dinv


def _prep(x_pad, w, degp):
    return pl.pallas_call(
        _prep_body,
        grid=(_NPAD // _RB,),
        in_specs=[
            pl.BlockSpec((_RB, _D), lambda i: (i, 0)),
            pl.BlockSpec((_D, _D), lambda i: (0, 0)),
            pl.BlockSpec((_RB, 1), lambda i: (i, 0)),
        ],
        out_specs=pl.BlockSpec((_RB, _D), lambda i: (i, 0)),
        out_shape=jax.ShapeDtypeStruct((_NPAD, _D), jnp.float32),
    )(x_pad, w, degp)


def _layer_body(agg_ref, hs_ref, degc_ref, b_ref, w_ref, out_ref):
    dinv = lax.rsqrt(degc_ref[...] + 1.0)
    z = (agg_ref[...] + hs_ref[...]) * dinv + b_ref[...]
    z = jnp.maximum(z, 0.0)
    out_ref[...] = jnp.dot(z, w_ref[...],
                           preferred_element_type=jnp.float32) * dinv


def _layer(agg, hs, degp, b, w):
    return pl.pallas_call(
        _layer_body,
        grid=(_NPAD // _RB,),
        in_specs=[
            pl.BlockSpec((_RB, _D), lambda i: (i, 0)),
            pl.BlockSpec((_RB, _D), lambda i: (i, 0)),
            pl.BlockSpec((_RB, 1), lambda i: (i, 0)),
            pl.BlockSpec((1, _D), lambda i: (0, 0)),
            pl.BlockSpec((_D, _D), lambda i: (0, 0)),
        ],
        out_specs=pl.BlockSpec((_RB, _D), lambda i: (i, 0)),
        out_shape=jax.ShapeDtypeStruct((_NPAD, _D), jnp.float32),
    )(agg, hs, degp, b, w)


def _final_body(agg_ref, hs_ref, degc_ref, b_ref, wfc_ref, bfc_ref,
                out_ref, acc_ref):
    i = pl.program_id(0)
    dinv = lax.rsqrt(degc_ref[...] + 1.0)
    z = (agg_ref[...] + hs_ref[...]) * dinv + b_ref[...]
    z = jnp.maximum(z, 0.0)
    rid = i * _RB + lax.broadcasted_iota(jnp.int32, (_RB, 1), 0)
    z = jnp.where(rid < _N, z, 0.0)

    @pl.when(i == 0)
    def _():
        acc_ref[...] = jnp.zeros_like(acc_ref)

    acc_ref[...] += jnp.sum(z, axis=0, keepdims=True)

    @pl.when(i == pl.num_programs(0) - 1)
    def _():
        g = acc_ref[...] * (1.0 / _N)
        out_ref[...] = jnp.dot(g, wfc_ref[...],
                               preferred_element_type=jnp.float32) + bfc_ref[...]


def _final(agg, hs, degp, b, wfc, bfc):
    return pl.pallas_call(
        _final_body,
        grid=(_NPAD // _RB,),
        in_specs=[
            pl.BlockSpec((_RB, _D), lambda i: (i, 0)),
            pl.BlockSpec((_RB, _D), lambda i: (i, 0)),
            pl.BlockSpec((_RB, 1), lambda i: (i, 0)),
            pl.BlockSpec((1, _D), lambda i: (0, 0)),
            pl.BlockSpec((_D, 1), lambda i: (0, 0)),
            pl.BlockSpec((1, 1), lambda i: (0, 0)),
        ],
        out_specs=pl.BlockSpec((1, 1), lambda i: (0, 0)),
        out_shape=jax.ShapeDtypeStruct((1, 1), jnp.float32),
        scratch_shapes=[pltpu.VMEM((1, _D), jnp.float32)],
    )(agg, hs, degp, b, wfc, bfc)


def kernel(x, edge_index, W1, b1, W2, b2, Wfc, bfc):
    src = edge_index[0]
    dst = edge_index[1]
    pad = (jnp.arange(_EPAD - _E, dtype=jnp.int32) % 128) + _N
    src3 = jnp.concatenate([src, pad]).reshape(_CHUNKS, _B)
    dstp = jnp.concatenate([dst, pad])
    garb = _NC_CORE + (dstp & (_GZ - 1))
    dloc = jnp.stack([
        jnp.where(dstp < _NC_CORE, dstp, garb),
        jnp.where(dstp >= _NC_CORE, dstp - _NC_CORE, garb),
    ]).reshape(2, _CHUNKS, _B)
    x_pad = jnp.zeros((_NPAD, _D), jnp.float32).at[:_N].set(x)
    zrow = jnp.zeros((_ZPT, _D), jnp.float32)
    ones = jnp.ones((_B, _D), jnp.float32)

    degc = _deg_kernel(dloc, ones, zrow)[:, :1]
    hs1 = _prep(x_pad, W1, degc)
    agg1 = _agg_kernel(hs1, src3, dloc, zrow)
    hs2 = _layer(agg1, hs1, degc, b1.reshape(1, _D), W2)
    agg2 = _agg_kernel(hs2, src3, dloc, zrow)
    out = _final(agg2, hs2, degc, b2.reshape(1, _D), Wfc, bfc.reshape(1, 1))
    return out.reshape((1,))


# speedup vs baseline: 17.4414x; 1.0899x over previous
import functools

import jax
import jax.numpy as jnp
from jax import lax
from jax.experimental import pallas as pl
from jax.experimental.pallas import tpu as pltpu
from jax.experimental.pallas import tpu_sc as plsc

_N = 10000
_E = 320000
_D = 128
_NPAD = 10240
_RPS = _NPAD // 16
_B = 128
_CPW = 80
_NBUF = 4
_GROUPS = _CPW // _NBUF
_NW = 32
_CHUNKS = _NW * _CPW
_EPAD = _CHUNKS * _B
_RB = 1024

_mesh = plsc.VectorSubcoreMesh(core_axis_name="c", subcore_axis_name="s")
_mesh1 = plsc.VectorSubcoreMesh(core_axis_name="c", subcore_axis_name="s",
                                num_cores=1)


_CPW_A = _CHUNKS // 16
_NQ = 4
_QCH = _CPW_A // _NQ
_NBUF_A = 4
_GROUPS_Q = _QCH // _NBUF_A
_NC_CORE = 5120
_GZ = 1024
_NLOC = _NC_CORE + _GZ
_ZPT = _NLOC // 16
_WPT = _NC_CORE // 16


@functools.partial(
    pl.kernel,
    out_type=jax.ShapeDtypeStruct((2, _NPAD, _D), jnp.float32),
    mesh=_mesh,
    scratch_types=[
        pltpu.VMEM((_CPW, _B), jnp.int32),
        pltpu.VMEM((_B, _D), jnp.float32),
        pltpu.VMEM_SHARED((_NPAD, _D), jnp.float32),
        pltpu.SemaphoreType.DMA((_NBUF,)),
    ],
)
def _deg_kernel(dst_hbm, ones_hbm, zrow_hbm, out_hbm, didx, ones_v, acc, sem):
    cid = lax.axis_index("c")
    sid = lax.axis_index("s")
    base = (cid * 16 + sid) * _CPW
    r0 = sid * _RPS
    pltpu.sync_copy(dst_hbm.at[pl.ds(base, _CPW)], didx)
    pltpu.sync_copy(ones_hbm, ones_v)
    pltpu.sync_copy(zrow_hbm, acc.at[pl.ds(r0, _RPS)])
    plsc.subcore_barrier()

    def body(g, carry):
        for b in range(_NBUF):
            jj = g * _NBUF + b
            pltpu.async_copy(ones_v, acc.at[didx.at[jj]], sem.at[b],
                             add=True)
        for b in range(_NBUF):
            jj = g * _NBUF + b
            pltpu.make_async_copy(ones_v, acc.at[didx.at[jj]],
                                  sem.at[b]).wait()
        return carry

    lax.fori_loop(0, _CPW // _NBUF, body, 0)
    plsc.subcore_barrier()
    pltpu.sync_copy(acc.at[pl.ds(r0, _RPS)],
                    out_hbm.at[cid].at[pl.ds(r0, _RPS)])


@functools.partial(
    pl.kernel,
    out_type=jax.ShapeDtypeStruct((_NPAD, _D), jnp.float32),
    mesh=_mesh,
    scratch_types=[
        pltpu.VMEM((_QCH, _B), jnp.int32),
        pltpu.VMEM((_QCH, _B), jnp.int32),
        pltpu.VMEM((_NBUF_A, _B, _D), jnp.float32),
        pltpu.VMEM_SHARED((_NLOC, _D), jnp.float32),
        pltpu.SemaphoreType.DMA((_NBUF_A,)),
        pltpu.SemaphoreType.DMA((_NBUF_A,)),
    ],
)
def _agg_kernel(hs_hbm, src_hbm, dloc_hbm, zrow_hbm, out_hbm,
                sidx, didx, rbuf, acc, gsem, ssem):
    cid = lax.axis_index("c")
    sid = lax.axis_index("s")
    base = sid * _CPW_A
    dlo = cid * _NC_CORE
    pltpu.sync_copy(zrow_hbm, acc.at[pl.ds(sid * _ZPT, _ZPT)])
    plsc.subcore_barrier()

    def quarter(q, carry):
        qb = base + q * _QCH
        pltpu.sync_copy(src_hbm.at[pl.ds(qb, _QCH)], sidx)
        pltpu.sync_copy(dloc_hbm.at[cid].at[pl.ds(qb, _QCH)], didx)

        for b in range(_NBUF_A):
            pltpu.async_copy(hs_hbm.at[sidx.at[b]], rbuf.at[b], gsem.at[b])

        def body(g, carry2):
            for b in range(_NBUF_A):
                jj = g * _NBUF_A + b
                pltpu.make_async_copy(hs_hbm.at[sidx.at[jj]], rbuf.at[b],
                                      gsem.at[b]).wait()
                pltpu.async_copy(rbuf.at[b], acc.at[didx.at[jj]], ssem.at[b],
                                 add=True)
            for b in range(_NBUF_A):
                jj = g * _NBUF_A + b
                pltpu.make_async_copy(rbuf.at[b], acc.at[didx.at[jj]],
                                      ssem.at[b]).wait()

                @pl.when(g < _GROUPS_Q - 1)
                def _(jj=jj, b=b):
                    pltpu.async_copy(hs_hbm.at[sidx.at[jj + _NBUF_A]],
                                     rbuf.at[b], gsem.at[b])
            return carry2

        lax.fori_loop(0, _GROUPS_Q, body, 0)
        return carry

    lax.fori_loop(0, _NQ, quarter, 0)
    plsc.subcore_barrier()
    pltpu.sync_copy(acc.at[pl.ds(sid * _WPT, _WPT)],
                    out_hbm.at[pl.ds(dlo + sid * _WPT, _WPT)])


def _prep_body(x_ref, w_ref, degp_ref, out_ref):
    dinv = lax.rsqrt(degp_ref[0] + degp_ref[1] + 1.0)
    h = jnp.dot(x_ref[...], w_ref[...],
                preferred_element_type=jnp.float32)
    out_ref[...] = h * dinv


def _prep(x_pad, w, degp):
    return pl.pallas_call(
        _prep_body,
        grid=(_NPAD // _RB,),
        in_specs=[
            pl.BlockSpec((_RB, _D), lambda i: (i, 0)),
            pl.BlockSpec((_D, _D), lambda i: (0, 0)),
            pl.BlockSpec((2, _RB, 1), lambda i: (0, i, 0)),
        ],
        out_specs=pl.BlockSpec((_RB, _D), lambda i: (i, 0)),
        out_shape=jax.ShapeDtypeStruct((_NPAD, _D), jnp.float32),
    )(x_pad, w, degp)


def _layer_body(agg_ref, hs_ref, degp_ref, b_ref, w_ref, out_ref):
    dinv = lax.rsqrt(degp_ref[0] + degp_ref[1] + 1.0)
    z = (agg_ref[...] + hs_ref[...]) * dinv + b_ref[...]
    z = jnp.maximum(z, 0.0)
    out_ref[...] = jnp.dot(z, w_ref[...],
                           preferred_element_type=jnp.float32) * dinv


def _layer(agg, hs, degp, b, w):
    return pl.pallas_call(
        _layer_body,
        grid=(_NPAD // _RB,),
        in_specs=[
            pl.BlockSpec((_RB, _D), lambda i: (i, 0)),
            pl.BlockSpec((_RB, _D), lambda i: (i, 0)),
            pl.BlockSpec((2, _RB, 1), lambda i: (0, i, 0)),
            pl.BlockSpec((1, _D), lambda i: (0, 0)),
            pl.BlockSpec((_D, _D), lambda i: (0, 0)),
        ],
        out_specs=pl.BlockSpec((_RB, _D), lambda i: (i, 0)),
        out_shape=jax.ShapeDtypeStruct((_NPAD, _D), jnp.float32),
    )(agg, hs, degp, b, w)


def _final_body(agg_ref, hs_ref, degp_ref, b_ref, wfc_ref, bfc_ref,
                out_ref, acc_ref):
    i = pl.program_id(0)
    dinv = lax.rsqrt(degp_ref[0] + degp_ref[1] + 1.0)
    z = (agg_ref[...] + hs_ref[...]) * dinv + b_ref[...]
    z = jnp.maximum(z, 0.0)
    rid = i * _RB + lax.broadcasted_iota(jnp.int32, (_RB, 1), 0)
    z = jnp.where(rid < _N, z, 0.0)

    @pl.when(i == 0)
    def _():
        acc_ref[...] = jnp.zeros_like(acc_ref)

    acc_ref[...] += jnp.sum(z, axis=0, keepdims=True)

    @pl.when(i == pl.num_programs(0) - 1)
    def _():
        g = acc_ref[...] * (1.0 / _N)
        out_ref[...] = jnp.dot(g, wfc_ref[...],
                               preferred_element_type=jnp.float32) + bfc_ref[...]


def _final(agg, hs, degp, b, wfc, bfc):
    return pl.pallas_call(
        _final_body,
        grid=(_NPAD // _RB,),
        in_specs=[
            pl.BlockSpec((_RB, _D), lambda i: (i, 0)),
            pl.BlockSpec((_RB, _D), lambda i: (i, 0)),
            pl.BlockSpec((2, _RB, 1), lambda i: (0, i, 0)),
            pl.BlockSpec((1, _D), lambda i: (0, 0)),
            pl.BlockSpec((_D, 1), lambda i: (0, 0)),
            pl.BlockSpec((1, 1), lambda i: (0, 0)),
        ],
        out_specs=pl.BlockSpec((1, 1), lambda i: (0, 0)),
        out_shape=jax.ShapeDtypeStruct((1, 1), jnp.float32),
        scratch_shapes=[pltpu.VMEM((1, _D), jnp.float32)],
    )(agg, hs, degp, b, wfc, bfc)


def kernel(x, edge_index, W1, b1, W2, b2, Wfc, bfc):
    src = edge_index[0]
    dst = edge_index[1]
    pad = (jnp.arange(_EPAD - _E, dtype=jnp.int32) % 128) + _N
    src3 = jnp.concatenate([src, pad]).reshape(_CHUNKS, _B)
    dstp = jnp.concatenate([dst, pad])
    garb = _NC_CORE + (dstp & (_GZ - 1))
    dloc = jnp.stack([
        jnp.where(dstp < _NC_CORE, dstp, garb),
        jnp.where(dstp >= _NC_CORE, dstp - _NC_CORE, garb),
    ]).reshape(2, _CHUNKS, _B)
    dst3 = dstp.reshape(_CHUNKS, _B)
    x_pad = jnp.zeros((_NPAD, _D), jnp.float32).at[:_N].set(x)
    zrow = jnp.zeros((_ZPT, _D), jnp.float32)
    zrow6 = jnp.zeros((_RPS, _D), jnp.float32)
    ones = jnp.ones((_B, _D), jnp.float32)

    degp = _deg_kernel(dst3, ones, zrow6)[:, :, :1]
    hs1 = _prep(x_pad, W1, degp)
    agg1 = _agg_kernel(hs1, src3, dloc, zrow)
    hs2 = _layer(agg1, hs1, degp, b1.reshape(1, _D), W2)
    agg2 = _agg_kernel(hs2, src3, dloc, zrow)
    out = _final(agg2, hs2, degp, b2.reshape(1, _D), Wfc, bfc.reshape(1, 1))
    return out.reshape((1,))
